# baseline (speedup 1.0000x reference)
import functools

import jax
import jax.numpy as jnp
from jax import lax
from jax.experimental import pallas as pl
from jax.experimental.pallas import tpu as pltpu
from jax.experimental.pallas import tpu_sc as plsc

F32 = jnp.float32

NC = 2
NS = 16
NW = NC * NS
CB = 128


def _leaky(x):
    return jnp.where(x >= 0, x, 0.01 * x)


def _dgT(x, w):
    return lax.dot_general(x, w, (((1,), (1,)), ((), ())),
                           preferred_element_type=F32)


_NSQ = 14


def _sigma_of(w):
    b = lax.dot_general(w, w, (((0,), (0,)), ((), ())),
                        preferred_element_type=F32)
    for _ in range(_NSQ):
        b = b / jnp.sqrt(jnp.sum(b * b))
        b = lax.dot_general(b, b, (((1,), (0,)), ((), ())),
                            preferred_element_type=F32)
    v = jnp.sum(b, axis=1, keepdims=True)
    wv = lax.dot_general(w, v, (((1,), (0,)), ((), ())),
                         preferred_element_type=F32)
    return jnp.sqrt(jnp.sum(wv * wv) / jnp.maximum(jnp.sum(v * v), 1e-30))


def _k0_body(w9_ref, wt_ref, s9_ref, st_ref):
    for i in range(9):
        s9_ref[i, :] = jnp.full((128,), _sigma_of(w9_ref[i]), F32)
    st_ref[0, :] = jnp.full((128,), _sigma_of(wt_ref[...]), F32)


def _k0(w9, wt):
    return pl.pallas_call(
        _k0_body,
        out_shape=[jax.ShapeDtypeStruct((9, 128), F32),
                   jax.ShapeDtypeStruct((1, 128), F32)],
    )(w9, wt)


def _k1_body(h_ref, p_ref, w1_ref, b1_ref, w2_ref, b2_ref, u_ref, v_ref):
    d = h_ref.shape[1]
    x = p_ref[...]
    hb = h_ref[...]
    u_ref[:, :d] = hb
    u_ref[:, d:] = _dgT(x, w2_ref[...]) + b2_ref[...]
    v_ref[:, :d] = hb
    v_ref[:, d:] = _dgT(x, w1_ref[...]) + b1_ref[...]


def _k1(h, p, w1, b1, w2, b2, bn):
    n, d = p.shape
    grid = n // bn
    wspec = pl.BlockSpec((d, d), lambda i: (0, 0))
    bspec = pl.BlockSpec((1, d), lambda i: (0, 0))
    nspec = pl.BlockSpec((bn, d), lambda i: (i, 0))
    ospec = pl.BlockSpec((bn, 2 * d), lambda i: (i, 0))
    return pl.pallas_call(
        _k1_body,
        grid=(grid,),
        in_specs=[nspec, nspec, wspec, bspec, wspec, bspec],
        out_specs=[ospec, ospec],
        out_shape=[jax.ShapeDtypeStruct((n, 2 * d), F32)] * 2,
    )(h, p, w1, b1, w2, b2)


CB2 = 64


def _k2(src, dst, u, v):
    e = src.shape[0]
    n, d2 = u.shape
    d = d2 // 2
    tch = e // CB2
    per_w = -(-tch // NW)
    half = (per_w + 1) // 2
    mesh = plsc.VectorSubcoreMesh(core_axis_name="c", subcore_axis_name="s",
                                  num_cores=NC, num_subcores=NS)

    @functools.partial(
        pl.kernel,
        out_type=jax.ShapeDtypeStruct((e, d2), F32),
        mesh=mesh,
        scratch_types=[
            [pltpu.VMEM((CB2,), jnp.int32)] * 2,
            [pltpu.VMEM((CB2,), jnp.int32)] * 2,
            [pltpu.VMEM((CB2, d2), F32)] * 2,
            [pltpu.VMEM((CB2, d2), F32)] * 2,
            [pltpu.VMEM((CB2, d2), F32)] * 2,
            [pltpu.SemaphoreType.DMA] * 2,
            pltpu.SemaphoreType.DMA,
        ],
    )
    def body(src_hbm, dst_hbm, u_hbm, v_hbm, ghh_out,
             idx_s, idx_d, ub, vb, ob, gsem, osem):
        wid = lax.axis_index("s") * NC + lax.axis_index("c")

        def gstart(b, j):
            base = j * CB2
            pltpu.sync_copy(src_hbm.at[pl.ds(base, CB2)], idx_s[b])
            pltpu.sync_copy(dst_hbm.at[pl.ds(base, CB2)], idx_d[b])
            pltpu.async_copy(u_hbm.at[idx_s[b]], ub[b], gsem[b])
            pltpu.async_copy(v_hbm.at[idx_d[b]], vb[b], gsem[b])

        def gwait(b):
            pltpu.make_async_copy(u_hbm.at[idx_s[b]], ub[b], gsem[b]).wait()
            pltpu.make_async_copy(v_hbm.at[idx_d[b]], vb[b], gsem[b]).wait()

        @pl.when(wid < tch)
        def _():
            gstart(0, wid)

        def pair(k2, carry):
            for b in (0, 1):
                k = 2 * k2 + b
                j = wid + k * NW
                nj = j + NW

                @pl.when(nj < tch)
                def _():
                    gstart(1 - b, nj)

                @pl.when(j < tch)
                def _():
                    gwait(b)

                    def row(r, rc):
                        for q in range(d // 16):
                            sl = pl.ds(q * 16, 16)
                            sh = pl.ds(d + q * 16, 16)
                            ob[b][r, sl] = vb[b][r, sh] - ub[b][r, sh]
                            ob[b][r, sh] = ub[b][r, sl] * vb[b][r, sl]
                        return rc

                    lax.fori_loop(0, CB2, row, 0, unroll=4)
                    base = j * CB2
                    c1 = pltpu.async_copy(ob[b], ghh_out.at[pl.ds(base, CB2)],
                                          osem)
                    c1.wait()

            return carry

        lax.fori_loop(0, half, pair, 0, unroll=False)

    return body(src, dst, u, v)


def _k3_body(e_ref, ghh_ref, we_ref, be_ref, wv_ref, bv_ref,
             eo_ref, gr_ref):
    d = e_ref.shape[1]
    e_b = e_ref[...]
    gh = ghh_ref[...]
    e1 = _dgT(e_b, we_ref[...]) + be_ref[...]
    en = 0.5 * (gh[:, :d] + e1)
    eo_ref[...] = e_b + _leaky(en)
    vj = jax.nn.relu(_dgT(gh[:, d:], wv_ref[...]) + bv_ref[...])
    sr = jnp.sum(vj, axis=1, keepdims=True)
    ss = jnp.sum(en * en, axis=1, keepdims=True)
    n2 = jnp.sqrt(ss)
    beta = jnp.exp(-0.5 * jnp.sqrt(n2) / 0.3)
    denom = 1.2 * n2 * jnp.sqrt(n2)
    coef = jnp.where(n2 > 1e-9, -sr * beta / denom, 0.0)
    gr_ref[...] = coef * en


def _k3_body_acc(e_ref, ghh_ref, we_ref, be_ref, wv_ref, bv_ref,
                 eop_ref, grp_ref, eo_ref, gr_ref):
    _k3_body(e_ref, ghh_ref, we_ref, be_ref, wv_ref, bv_ref,
             eo_ref, gr_ref)


def _k3(e, ghh_i, we, be, wv, bv, be_rows, chunk_off, prev):
    ne, d = e.shape
    ce = ghh_i.shape[0]
    grid = ce // be_rows
    off = chunk_off
    fspec = pl.BlockSpec((be_rows, d), lambda i: (off + i, 0))
    cspec = pl.BlockSpec((be_rows, 2 * d), lambda i: (i, 0))
    wspec = pl.BlockSpec((d, d), lambda i: (0, 0))
    bspec = pl.BlockSpec((1, d), lambda i: (0, 0))
    aspec = pl.BlockSpec(memory_space=pl.ANY)
    out_shape = [jax.ShapeDtypeStruct((ne, d), F32)] * 2
    out_specs = [fspec, fspec]
    if prev is None:
        return pl.pallas_call(
            _k3_body,
            grid=(grid,),
            in_specs=[fspec, cspec, wspec, bspec, wspec, bspec],
            out_specs=out_specs,
            out_shape=out_shape,
        )(e, ghh_i, we, be, wv, bv)
    return pl.pallas_call(
        _k3_body_acc,
        grid=(grid,),
        in_specs=[fspec, cspec, wspec, bspec, wspec, bspec,
                  aspec, aspec],
        out_specs=out_specs,
        out_shape=out_shape,
        input_output_aliases={6: 0, 7: 1},
    )(e, ghh_i, we, be, wv, bv, prev[0], prev[1])


def _k4(grad, dst_g, dst, tm, n):
    e, d = grad.shape
    npad = 10240
    tch = e // CB
    per_core = tch // NC
    per_w = -(-per_core // NS)
    rps = npad // NS
    zr = 128
    mesh = plsc.VectorSubcoreMesh(core_axis_name="c", subcore_axis_name="s",
                                  num_cores=NC, num_subcores=NS)

    @functools.partial(
        pl.kernel,
        out_type=(jax.ShapeDtypeStruct((NC, npad, d), F32),
                  jax.ShapeDtypeStruct((NC, npad), F32)),
        mesh=mesh,
        scratch_types=[
            [pltpu.VMEM((CB,), jnp.int32)] * 2,
            [pltpu.VMEM((CB,), jnp.int32)] * 2,
            [pltpu.VMEM((CB, d), F32)] * 2,
            [pltpu.VMEM((CB,), F32)] * 2,
            pltpu.VMEM((rps,), F32),
            pltpu.VMEM_SHARED((npad, d), F32),
            pltpu.VMEM_SHARED((npad,), F32),
            [pltpu.SemaphoreType.DMA] * 2,
        ],
    )
    def body(grad_hbm, dstg_hbm, dst_hbm, tm_hbm, fab_out, cnt_out,
             idx, idx_t, gbuf, tbuf, zvec, fab_sp, cnt_sp, isem):
        zbuf = gbuf[0]
        cid = lax.axis_index("c")
        sid = lax.axis_index("s")

        def zrow(r, rc):
            for q in range(d // 16):
                zbuf[r, pl.ds(q * 16, 16)] = jnp.zeros((16,), F32)
            return rc

        lax.fori_loop(0, zr, zrow, 0, unroll=False)

        def zvecrow(r, rc):
            zvec[pl.ds(r * 16, 16)] = jnp.zeros((16,), F32)
            return rc

        lax.fori_loop(0, rps // 16, zvecrow, 0, unroll=False)

        for q in range(rps // zr):
            pltpu.sync_copy(zbuf, fab_sp.at[pl.ds(sid * rps + q * zr, zr)])
        pltpu.sync_copy(zvec, cnt_sp.at[pl.ds(sid * rps, rps)])
        plsc.subcore_barrier()

        j0 = cid * per_core + sid
        lim = (cid + 1) * per_core

        def istart(b, j):
            base = j * CB
            pltpu.async_copy(dstg_hbm.at[pl.ds(base, CB)], idx[b], isem[b])
            pltpu.async_copy(dst_hbm.at[pl.ds(base, CB)], idx_t[b], isem[b])
            pltpu.async_copy(grad_hbm.at[pl.ds(base, CB)], gbuf[b], isem[b])
            pltpu.async_copy(tm_hbm.at[pl.ds(base, CB)], tbuf[b], isem[b])

        def iwait(b):
            pltpu.make_async_copy(dstg_hbm.at[pl.ds(0, CB)], idx[b],
                                  isem[b]).wait()
            pltpu.make_async_copy(dst_hbm.at[pl.ds(0, CB)], idx_t[b],
                                  isem[b]).wait()
            pltpu.make_async_copy(grad_hbm.at[pl.ds(0, CB)], gbuf[b],
                                  isem[b]).wait()
            pltpu.make_async_copy(tm_hbm.at[pl.ds(0, CB)], tbuf[b],
                                  isem[b]).wait()

        @pl.when(j0 < lim)
        def _():
            istart(0, j0)

        def pair(k2, carry):
            for b in (0, 1):
                k = 2 * k2 + b
                j = j0 + k * NS
                nj = j + NS

                @pl.when(nj < lim)
                def _():
                    istart(1 - b, nj)

                @pl.when(j < lim)
                def _():
                    iwait(b)
                    pltpu.sync_copy(gbuf[b], fab_sp.at[idx[b]], add=True)
                    pltpu.sync_copy(tbuf[b], cnt_sp.at[idx_t[b]], add=True)

            return carry

        lax.fori_loop(0, (per_w + 1) // 2, pair, 0, unroll=False)
        plsc.subcore_barrier()

        for q in range(rps // zr):
            r0 = sid * rps + q * zr
            pltpu.sync_copy(fab_sp.at[pl.ds(r0, zr)], zbuf)
            pltpu.sync_copy(zbuf, fab_out.at[cid, pl.ds(r0, zr)])
        pltpu.sync_copy(cnt_sp.at[pl.ds(sid * rps, rps)], zvec)
        pltpu.sync_copy(zvec, cnt_out.at[cid, pl.ds(sid * rps, rps)])

    return body(grad, dst_g, dst, tm)


def _k5_body(h_ref, p_ref, d_ref, dt_ref, cnt_ref, fab_ref,
             wv1_ref, bv1_ref, wv2_ref, bv2_ref, wp3_ref, bp3_ref,
             wd1_ref, bd1_ref, wd2_ref, bd2_ref, wth_ref, wtd_ref, bt_ref,
             ho_ref, po_ref, do_ref):
    h_b = h_ref[...]
    p_b = p_ref[...]
    d_b = d_ref[...]
    v1 = _dgT(h_b, wv1_ref[...]) + bv1_ref[...]
    v2 = _dgT(h_b, wv2_ref[...]) + bv2_ref[...]
    p3 = _dgT(p_b, wp3_ref[...]) + bp3_ref[...]
    d1 = _dgT(d_b, wd1_ref[...]) + bd1_ref[...]
    d2 = _dgT(d_b, wd2_ref[...]) + bd2_ref[...]
    tau = jax.nn.relu(_dgT(h_b, wth_ref[...]) + _dgT(d_b, wtd_ref[...])
                      + bt_ref[...])
    a = d1 - v1
    f = a * tau * cnt_ref[...] - (fab_ref[0] + fab_ref[1])
    dtc = dt_ref[...]
    fdt = f * dtc
    ho_ref[...] = h_b + _leaky(v2 + fdt)
    po_ref[...] = p_b + _leaky(p3 + fdt + 0.5 * fdt * dtc)
    fn = jnp.sqrt(jnp.sum(f * f, axis=1, keepdims=True)) + 1e-9
    do_ref[...] = d_b + _leaky(d2 + f / fn)


def _k5(h, p, d, dt_col, cnt_col, fab_p, ws, bn):
    n, dd = h.shape
    grid = n // bn
    nspec = pl.BlockSpec((bn, dd), lambda i: (i, 0))
    cspec = pl.BlockSpec((bn, 1), lambda i: (i, 0))
    fspec = pl.BlockSpec((2, bn, dd), lambda i: (0, i, 0))
    wspec = pl.BlockSpec((dd, dd), lambda i: (0, 0))
    bspec = pl.BlockSpec((1, dd), lambda i: (0, 0))
    wb = []
    specs = []
    for (w, b) in ws[:-1]:
        wb += [w, b]
        specs += [wspec, bspec]
    wth, wtd, bt = ws[-1]
    wb += [wth, wtd, bt]
    specs += [wspec, wspec, bspec]
    return pl.pallas_call(
        _k5_body,
        grid=(grid,),
        in_specs=[nspec, nspec, nspec, cspec, cspec, fspec] + specs,
        out_specs=[nspec, nspec, nspec],
        out_shape=[jax.ShapeDtypeStruct((n, dd), F32)] * 3,
    )(h, p, d, dt_col, cnt_col, fab_p, *wb)


def kernel(h, e, p, d, dt, edge_index, spatial_mask,
           W_V1, b_V1, W_V2, b_V2, W_E1, b_E1,
           W_P1, b_P1, W_P2, b_P2, W_P3, b_P3,
           W_D1, b_D1, W_D2, b_D2, W_V, b_V, W_T, b_T):
    n, dd = h.shape

    w9 = jnp.stack([W_V1, W_V2, W_E1, W_P1, W_P2, W_P3, W_D1, W_D2, W_V])
    s9, st = _k0(w9, W_T)
    sig = s9[:, 0]
    wv1, wv2, we1 = W_V1 / sig[0], W_V2 / sig[1], W_E1 / sig[2]
    wp1, wp2, wp3 = W_P1 / sig[3], W_P2 / sig[4], W_P3 / sig[5]
    wd1, wd2, wv = W_D1 / sig[6], W_D2 / sig[7], W_V / sig[8]
    wt = W_T / st[0, 0]
    wth, wtd = wt[:, :dd], wt[:, dd:]

    row = lambda b: b.reshape(1, dd)
    src = edge_index[0]
    dst = edge_index[1]
    dst_g = jnp.where(spatial_mask == 1, dst, jnp.int32(10239))
    tm = (spatial_mask == 0).astype(F32)
    dt_col = dt[:, None]

    u, v = _k1(h, p, wp1, row(b_P1), wp2, row(b_P2), bn=1000)

    bounds = [0, 80000, 176000, 272000, 320000]
    prev = None
    for i in range(len(bounds) - 1):
        sl = slice(bounds[i], bounds[i + 1])
        ghh_i = _k2(src[sl], dst[sl], u, v)
        prev = _k3(e, ghh_i, we1, row(b_E1), wv, row(b_V),
                   be_rows=2000, chunk_off=bounds[i] // 2000, prev=prev)
    e_out, grad = prev
    fab_p, cnt_p = _k4(grad, dst_g, dst, tm, n)
    cnt_col = (cnt_p[0] + cnt_p[1])[:n, None]
    ws = [(wv1, row(b_V1)), (wv2, row(b_V2)), (wp3, row(b_P3)),
          (wd1, row(b_D1)), (wd2, row(b_D2)), (wth, wtd, row(b_T))]
    h_out, p_out, d_out = _k5(h, p, d, dt_col, cnt_col, fab_p, ws, bn=1000)
    return h_out, e_out, p_out, d_out


# speedup vs baseline: 1.2354x; 1.2354x over previous
import functools

import jax
import jax.numpy as jnp
from jax import lax
from jax.experimental import pallas as pl
from jax.experimental.pallas import tpu as pltpu
from jax.experimental.pallas import tpu_sc as plsc

F32 = jnp.float32

NC = 2
NS = 16
NW = NC * NS
CB = 128


def _leaky(x):
    return jnp.where(x >= 0, x, 0.01 * x)


def _dgT(x, w):
    return lax.dot_general(x, w, (((1,), (1,)), ((), ())),
                           preferred_element_type=F32)


_NSQ = 14


def _sigma_of(w):
    b = lax.dot_general(w, w, (((0,), (0,)), ((), ())),
                        preferred_element_type=F32)
    for _ in range(_NSQ):
        b = b / jnp.sqrt(jnp.sum(b * b))
        b = lax.dot_general(b, b, (((1,), (0,)), ((), ())),
                            preferred_element_type=F32)
    v = jnp.sum(b, axis=1, keepdims=True)
    wv = lax.dot_general(w, v, (((1,), (0,)), ((), ())),
                         preferred_element_type=F32)
    return jnp.sqrt(jnp.sum(wv * wv) / jnp.maximum(jnp.sum(v * v), 1e-30))


def _k0_body(w9_ref, wt_ref, s9_ref, st_ref):
    for i in range(9):
        s9_ref[i, :] = jnp.full((128,), _sigma_of(w9_ref[i]), F32)
    st_ref[0, :] = jnp.full((128,), _sigma_of(wt_ref[...]), F32)


def _k0(w9, wt):
    return pl.pallas_call(
        _k0_body,
        out_shape=[jax.ShapeDtypeStruct((9, 128), F32),
                   jax.ShapeDtypeStruct((1, 128), F32)],
    )(w9, wt)


def _k1_body(h_ref, p_ref, w1_ref, b1_ref, w2_ref, b2_ref, u_ref, v_ref):
    d = h_ref.shape[1]
    x = p_ref[...]
    hb = h_ref[...]
    u_ref[:, :d] = hb
    u_ref[:, d:] = _dgT(x, w2_ref[...]) + b2_ref[...]
    v_ref[:, :d] = hb
    v_ref[:, d:] = _dgT(x, w1_ref[...]) + b1_ref[...]


def _k1(h, p, w1, b1, w2, b2, bn):
    n, d = p.shape
    grid = n // bn
    wspec = pl.BlockSpec((d, d), lambda i: (0, 0))
    bspec = pl.BlockSpec((1, d), lambda i: (0, 0))
    nspec = pl.BlockSpec((bn, d), lambda i: (i, 0))
    ospec = pl.BlockSpec((bn, 2 * d), lambda i: (i, 0))
    return pl.pallas_call(
        _k1_body,
        grid=(grid,),
        in_specs=[nspec, nspec, wspec, bspec, wspec, bspec],
        out_specs=[ospec, ospec],
        out_shape=[jax.ShapeDtypeStruct((n, 2 * d), F32)] * 2,
    )(h, p, w1, b1, w2, b2)


CB2 = 64


def _k2(src, dst, u, v):
    e = src.shape[0]
    n, d2 = u.shape
    d = d2 // 2
    tch = e // CB2
    per_w = -(-tch // NW)
    half = (per_w + 1) // 2
    mesh = plsc.VectorSubcoreMesh(core_axis_name="c", subcore_axis_name="s",
                                  num_cores=NC, num_subcores=NS)

    @functools.partial(
        pl.kernel,
        out_type=jax.ShapeDtypeStruct((e, d2), F32),
        mesh=mesh,
        scratch_types=[
            [pltpu.VMEM((CB2,), jnp.int32)] * 2,
            [pltpu.VMEM((CB2,), jnp.int32)] * 2,
            [pltpu.VMEM((CB2, d2), F32)] * 2,
            [pltpu.VMEM((CB2, d2), F32)] * 2,
            [pltpu.VMEM((CB2, d2), F32)] * 2,
            [pltpu.SemaphoreType.DMA] * 2,
            pltpu.SemaphoreType.DMA,
        ],
    )
    def body(src_hbm, dst_hbm, u_hbm, v_hbm, ghh_out,
             idx_s, idx_d, ub, vb, ob, gsem, osem):
        wid = lax.axis_index("s") * NC + lax.axis_index("c")

        def gstart(b, j):
            base = j * CB2
            pltpu.sync_copy(src_hbm.at[pl.ds(base, CB2)], idx_s[b])
            pltpu.sync_copy(dst_hbm.at[pl.ds(base, CB2)], idx_d[b])
            pltpu.async_copy(u_hbm.at[idx_s[b]], ub[b], gsem[b])
            pltpu.async_copy(v_hbm.at[idx_d[b]], vb[b], gsem[b])

        def gwait(b):
            pltpu.make_async_copy(u_hbm.at[idx_s[b]], ub[b], gsem[b]).wait()
            pltpu.make_async_copy(v_hbm.at[idx_d[b]], vb[b], gsem[b]).wait()

        @pl.when(wid < tch)
        def _():
            gstart(0, wid)

        def pair(k2, carry):
            for b in (0, 1):
                k = 2 * k2 + b
                j = wid + k * NW
                nj = j + NW

                @pl.when(nj < tch)
                def _():
                    gstart(1 - b, nj)

                @pl.when(j < tch)
                def _():
                    gwait(b)

                    def row(r, rc):
                        for q in range(d // 16):
                            sl = pl.ds(q * 16, 16)
                            sh = pl.ds(d + q * 16, 16)
                            ob[b][r, sl] = vb[b][r, sh] - ub[b][r, sh]
                            ob[b][r, sh] = ub[b][r, sl] * vb[b][r, sl]
                        return rc

                    lax.fori_loop(0, CB2, row, 0, unroll=False)
                    base = j * CB2
                    c1 = pltpu.async_copy(ob[b], ghh_out.at[pl.ds(base, CB2)],
                                          osem)
                    c1.wait()

            return carry

        lax.fori_loop(0, half, pair, 0, unroll=False)

    return body(src, dst, u, v)


def _k3_body(e_ref, ghh_ref, we_ref, be_ref, wv_ref, bv_ref,
             eo_ref, gr_ref):
    d = e_ref.shape[1]
    e_b = e_ref[...]
    gh = ghh_ref[...]
    e1 = _dgT(e_b, we_ref[...]) + be_ref[...]
    en = 0.5 * (gh[:, :d] + e1)
    eo_ref[...] = e_b + _leaky(en)
    vj = jax.nn.relu(_dgT(gh[:, d:], wv_ref[...]) + bv_ref[...])
    sr = jnp.sum(vj, axis=1, keepdims=True)
    ss = jnp.sum(en * en, axis=1, keepdims=True)
    n2 = jnp.sqrt(ss)
    beta = jnp.exp(-0.5 * jnp.sqrt(n2) / 0.3)
    denom = 1.2 * n2 * jnp.sqrt(n2)
    coef = jnp.where(n2 > 1e-9, -sr * beta / denom, 0.0)
    gr_ref[...] = coef * en


def _k3_body_acc(e_ref, ghh_ref, we_ref, be_ref, wv_ref, bv_ref,
                 eop_ref, grp_ref, eo_ref, gr_ref):
    _k3_body(e_ref, ghh_ref, we_ref, be_ref, wv_ref, bv_ref,
             eo_ref, gr_ref)


def _k3(e, ghh_i, we, be, wv, bv, be_rows, chunk_off, prev):
    ne, d = e.shape
    ce = ghh_i.shape[0]
    grid = ce // be_rows
    off = chunk_off
    fspec = pl.BlockSpec((be_rows, d), lambda i: (off + i, 0))
    cspec = pl.BlockSpec((be_rows, 2 * d), lambda i: (i, 0))
    wspec = pl.BlockSpec((d, d), lambda i: (0, 0))
    bspec = pl.BlockSpec((1, d), lambda i: (0, 0))
    aspec = pl.BlockSpec(memory_space=pl.ANY)
    out_shape = [jax.ShapeDtypeStruct((ne, d), F32)] * 2
    out_specs = [fspec, fspec]
    if prev is None:
        return pl.pallas_call(
            _k3_body,
            grid=(grid,),
            in_specs=[fspec, cspec, wspec, bspec, wspec, bspec],
            out_specs=out_specs,
            out_shape=out_shape,
        )(e, ghh_i, we, be, wv, bv)
    return pl.pallas_call(
        _k3_body_acc,
        grid=(grid,),
        in_specs=[fspec, cspec, wspec, bspec, wspec, bspec,
                  aspec, aspec],
        out_specs=out_specs,
        out_shape=out_shape,
        input_output_aliases={6: 0, 7: 1},
    )(e, ghh_i, we, be, wv, bv, prev[0], prev[1])


def _k4(grad, dst_g, dst, tm, n):
    e, d = grad.shape
    npad = 10240
    tch = e // CB
    per_core = tch // NC
    per_w = -(-per_core // NS)
    rps = npad // NS
    zr = 128
    mesh = plsc.VectorSubcoreMesh(core_axis_name="c", subcore_axis_name="s",
                                  num_cores=NC, num_subcores=NS)

    @functools.partial(
        pl.kernel,
        out_type=(jax.ShapeDtypeStruct((NC, npad, d), F32),
                  jax.ShapeDtypeStruct((NC, npad), F32)),
        mesh=mesh,
        scratch_types=[
            [pltpu.VMEM((CB,), jnp.int32)] * 2,
            [pltpu.VMEM((CB,), jnp.int32)] * 2,
            [pltpu.VMEM((CB, d), F32)] * 2,
            [pltpu.VMEM((CB,), F32)] * 2,
            pltpu.VMEM((rps,), F32),
            pltpu.VMEM_SHARED((npad, d), F32),
            pltpu.VMEM_SHARED((npad,), F32),
            [pltpu.SemaphoreType.DMA] * 2,
        ],
    )
    def body(grad_hbm, dstg_hbm, dst_hbm, tm_hbm, fab_out, cnt_out,
             idx, idx_t, gbuf, tbuf, zvec, fab_sp, cnt_sp, isem):
        zbuf = gbuf[0]
        cid = lax.axis_index("c")
        sid = lax.axis_index("s")

        def zrow(r, rc):
            for q in range(d // 16):
                zbuf[r, pl.ds(q * 16, 16)] = jnp.zeros((16,), F32)
            return rc

        lax.fori_loop(0, zr, zrow, 0, unroll=False)

        def zvecrow(r, rc):
            zvec[pl.ds(r * 16, 16)] = jnp.zeros((16,), F32)
            return rc

        lax.fori_loop(0, rps // 16, zvecrow, 0, unroll=False)

        for q in range(rps // zr):
            pltpu.sync_copy(zbuf, fab_sp.at[pl.ds(sid * rps + q * zr, zr)])
        pltpu.sync_copy(zvec, cnt_sp.at[pl.ds(sid * rps, rps)])
        plsc.subcore_barrier()

        j0 = cid * per_core + sid
        lim = (cid + 1) * per_core

        def istart(b, j):
            base = j * CB
            pltpu.async_copy(dstg_hbm.at[pl.ds(base, CB)], idx[b], isem[b])
            pltpu.async_copy(dst_hbm.at[pl.ds(base, CB)], idx_t[b], isem[b])
            pltpu.async_copy(grad_hbm.at[pl.ds(base, CB)], gbuf[b], isem[b])
            pltpu.async_copy(tm_hbm.at[pl.ds(base, CB)], tbuf[b], isem[b])

        def iwait(b):
            pltpu.make_async_copy(dstg_hbm.at[pl.ds(0, CB)], idx[b],
                                  isem[b]).wait()
            pltpu.make_async_copy(dst_hbm.at[pl.ds(0, CB)], idx_t[b],
                                  isem[b]).wait()
            pltpu.make_async_copy(grad_hbm.at[pl.ds(0, CB)], gbuf[b],
                                  isem[b]).wait()
            pltpu.make_async_copy(tm_hbm.at[pl.ds(0, CB)], tbuf[b],
                                  isem[b]).wait()

        @pl.when(j0 < lim)
        def _():
            istart(0, j0)

        def pair(k2, carry):
            for b in (0, 1):
                k = 2 * k2 + b
                j = j0 + k * NS
                nj = j + NS

                @pl.when(nj < lim)
                def _():
                    istart(1 - b, nj)

                @pl.when(j < lim)
                def _():
                    iwait(b)
                    pltpu.sync_copy(gbuf[b], fab_sp.at[idx[b]], add=True)
                    pltpu.sync_copy(tbuf[b], cnt_sp.at[idx_t[b]], add=True)

            return carry

        lax.fori_loop(0, (per_w + 1) // 2, pair, 0, unroll=False)
        plsc.subcore_barrier()

        for q in range(rps // zr):
            r0 = sid * rps + q * zr
            pltpu.sync_copy(fab_sp.at[pl.ds(r0, zr)], zbuf)
            pltpu.sync_copy(zbuf, fab_out.at[cid, pl.ds(r0, zr)])
        pltpu.sync_copy(cnt_sp.at[pl.ds(sid * rps, rps)], zvec)
        pltpu.sync_copy(zvec, cnt_out.at[cid, pl.ds(sid * rps, rps)])

    return body(grad, dst_g, dst, tm)


def _k5_body(h_ref, p_ref, d_ref, dt_ref, cnt_ref, fab_ref,
             wv1_ref, bv1_ref, wv2_ref, bv2_ref, wp3_ref, bp3_ref,
             wd1_ref, bd1_ref, wd2_ref, bd2_ref, wth_ref, wtd_ref, bt_ref,
             ho_ref, po_ref, do_ref):
    h_b = h_ref[...]
    p_b = p_ref[...]
    d_b = d_ref[...]
    v1 = _dgT(h_b, wv1_ref[...]) + bv1_ref[...]
    v2 = _dgT(h_b, wv2_ref[...]) + bv2_ref[...]
    p3 = _dgT(p_b, wp3_ref[...]) + bp3_ref[...]
    d1 = _dgT(d_b, wd1_ref[...]) + bd1_ref[...]
    d2 = _dgT(d_b, wd2_ref[...]) + bd2_ref[...]
    tau = jax.nn.relu(_dgT(h_b, wth_ref[...]) + _dgT(d_b, wtd_ref[...])
                      + bt_ref[...])
    a = d1 - v1
    f = a * tau * cnt_ref[...] - (fab_ref[0] + fab_ref[1])
    dtc = dt_ref[...]
    fdt = f * dtc
    ho_ref[...] = h_b + _leaky(v2 + fdt)
    po_ref[...] = p_b + _leaky(p3 + fdt + 0.5 * fdt * dtc)
    fn = jnp.sqrt(jnp.sum(f * f, axis=1, keepdims=True)) + 1e-9
    do_ref[...] = d_b + _leaky(d2 + f / fn)


def _k5(h, p, d, dt_col, cnt_col, fab_p, ws, bn):
    n, dd = h.shape
    grid = n // bn
    nspec = pl.BlockSpec((bn, dd), lambda i: (i, 0))
    cspec = pl.BlockSpec((bn, 1), lambda i: (i, 0))
    fspec = pl.BlockSpec((2, bn, dd), lambda i: (0, i, 0))
    wspec = pl.BlockSpec((dd, dd), lambda i: (0, 0))
    bspec = pl.BlockSpec((1, dd), lambda i: (0, 0))
    wb = []
    specs = []
    for (w, b) in ws[:-1]:
        wb += [w, b]
        specs += [wspec, bspec]
    wth, wtd, bt = ws[-1]
    wb += [wth, wtd, bt]
    specs += [wspec, wspec, bspec]
    return pl.pallas_call(
        _k5_body,
        grid=(grid,),
        in_specs=[nspec, nspec, nspec, cspec, cspec, fspec] + specs,
        out_specs=[nspec, nspec, nspec],
        out_shape=[jax.ShapeDtypeStruct((n, dd), F32)] * 3,
    )(h, p, d, dt_col, cnt_col, fab_p, *wb)


def kernel(h, e, p, d, dt, edge_index, spatial_mask,
           W_V1, b_V1, W_V2, b_V2, W_E1, b_E1,
           W_P1, b_P1, W_P2, b_P2, W_P3, b_P3,
           W_D1, b_D1, W_D2, b_D2, W_V, b_V, W_T, b_T):
    n, dd = h.shape

    w9 = jnp.stack([W_V1, W_V2, W_E1, W_P1, W_P2, W_P3, W_D1, W_D2, W_V])
    s9, st = _k0(w9, W_T)
    sig = s9[:, 0]
    wv1, wv2, we1 = W_V1 / sig[0], W_V2 / sig[1], W_E1 / sig[2]
    wp1, wp2, wp3 = W_P1 / sig[3], W_P2 / sig[4], W_P3 / sig[5]
    wd1, wd2, wv = W_D1 / sig[6], W_D2 / sig[7], W_V / sig[8]
    wt = W_T / st[0, 0]
    wth, wtd = wt[:, :dd], wt[:, dd:]

    row = lambda b: b.reshape(1, dd)
    src = edge_index[0]
    dst = edge_index[1]
    dst_g = jnp.where(spatial_mask == 1, dst, jnp.int32(10239))
    tm = (spatial_mask == 0).astype(F32)
    dt_col = dt[:, None]

    u, v = _k1(h, p, wp1, row(b_P1), wp2, row(b_P2), bn=1000)

    bounds = [0, 80000, 176000, 272000, 320000]
    prev = None
    for i in range(len(bounds) - 1):
        sl = slice(bounds[i], bounds[i + 1])
        ghh_i = _k2(src[sl], dst[sl], u, v)
        prev = _k3(e, ghh_i, we1, row(b_E1), wv, row(b_V),
                   be_rows=2000, chunk_off=bounds[i] // 2000, prev=prev)
    e_out, grad = prev
    fab_p, cnt_p = _k4(grad, dst_g, dst, tm, n)
    cnt_col = (cnt_p[0] + cnt_p[1])[:n, None]
    ws = [(wv1, row(b_V1)), (wv2, row(b_V2)), (wp3, row(b_P3)),
          (wd1, row(b_D1)), (wd2, row(b_D2)), (wth, wtd, row(b_T))]
    h_out, p_out, d_out = _k5(h, p, d, dt_col, cnt_col, fab_p, ws, bn=1000)
    return h_out, e_out, p_out, d_out
